# Initial kernel scaffold; baseline (speedup 1.0000x reference)
import jax
import jax.numpy as jnp
from jax.experimental import pallas as pl


def kernel(H, edge_index, edge_attr, module_id, Wq, Wk, Wv, Wo, w_a, b_a, ln1_g, ln1_b, W1, b1, W2, b2, ln2_g, ln2_b):
    raise NotImplementedError("write your pallas kernel here")


# speedup vs baseline: 88.0542x; 88.0542x over previous
import functools
import math

import jax
import jax.numpy as jnp
import numpy as np
from jax import lax
from jax.experimental import pallas as pl
from jax.experimental.pallas import tpu as pltpu
from jax.experimental.pallas import tpu_sc as plsc

N = 10000
E = 320000
D = 128
NH = 4
AD = 32
INV_SQRT_AD = 1.0 / math.sqrt(AD)

NC = 2
NS = 16
NW = NC * NS
EPW = E // NW
C1 = 80
NCHUNK = EPW // C1
C2 = 48
SEG = 80
SPILL_W = EPW + 2 * SEG
MODP = 2504
N_PAD = 10240
ROWS_PT = N_PAD // NS
NEX = N_PAD // 8
ROWS_EX_PT = NEX // NS

_DEBUG_PHASE1 = True
_DEBUG_PHASE2 = True
_DBG = 4
_DBG_EXTAB = False
f32 = jnp.float32
i32 = jnp.int32


def _qkv_body(h_ref, w_ref, q_ref, kv_ref):
    qkv = jnp.dot(h_ref[...], w_ref[...], preferred_element_type=f32)
    q_ref[...] = qkv[:, :D]
    kv_ref[...] = qkv[:, D:]


_qkv_call = pl.pallas_call(
    _qkv_body,
    grid=(10,),
    in_specs=[
        pl.BlockSpec((N // 10, D), lambda i: (i, 0)),
        pl.BlockSpec((D, 3 * D), lambda i: (0, 0)),
    ],
    out_specs=[
        pl.BlockSpec((N // 10, D), lambda i: (i, 0)),
        pl.BlockSpec((N // 10, 2 * D), lambda i: (i, 0)),
    ],
    out_shape=[
        jax.ShapeDtypeStruct((N, D), f32),
        jax.ShapeDtypeStruct((N, 2 * D), f32),
    ],
)


def _edge_body(
    qtab, kvtab, src_h, dst_h, w_h, midp_h, wab_h,
    omsg_h, oex_h, ssp_h, dsp_h, wsp_h,
    srcv, dstv, wv, midpv,
    stg_s, stg_d, stg_w, sidx, didx, sidx2, sidx8, wv2,
    qb, kvb, msgb, exb, wabv,
    msgtab, extab,
    sem,
):
    cid = lax.axis_index("c")
    sid = lax.axis_index("s")
    wid = cid * NS + sid
    ebase = wid * EPW
    lane = lax.iota(i32, 16)
    lane4 = lane < 4
    z16i = jnp.zeros((16,), i32)
    z16f = jnp.zeros((16,), f32)

    rbase = pl.multiple_of(sid * ROWS_PT, 8)
    if True:
        def zrow(r, _):
            for k in range(D // 16):
                msgb[r, pl.ds(k * 16, 16)] = z16f
                exb[r, pl.ds(k * 16, 16)] = z16f
            return 0
        lax.fori_loop(0, C2, zrow, 0)

        def zcp(t, _):
            ro = pl.multiple_of(rbase + t * 40, 8)
            pltpu.sync_copy(msgb.at[pl.ds(0, 40)], msgtab.at[pl.ds(ro, 40)])
            return 0
        lax.fori_loop(0, ROWS_PT // 40, zcp, 0)
        rbase2 = pl.multiple_of(sid * ROWS_EX_PT, 8)

        def zcp2(t, _):
            ro = pl.multiple_of(rbase2 + t * 40, 8)
            pltpu.sync_copy(exb.at[pl.ds(0, 40)], extab.at[pl.ds(ro, 40)])
            return 0
        lax.fori_loop(0, ROWS_EX_PT // 40, zcp2, 0)
    pltpu.sync_copy(wab_h, wabv)
    pltpu.sync_copy(midp_h, midpv)
    wav = wabv[0, :]
    bav = wabv[1, :]

    sbase = pl.multiple_of(wid * SPILL_W, 8)

    def _flush(staged, nseg):
        off = pl.multiple_of(nseg * SEG, 8)
        pltpu.sync_copy(stg_s.at[pl.ds(0, SEG)], ssp_h.at[pl.ds(sbase + off, SEG)])
        pltpu.sync_copy(stg_d.at[pl.ds(0, SEG)], dsp_h.at[pl.ds(sbase + off, SEG)])
        pltpu.sync_copy(stg_w.at[pl.ds(0, SEG)], wsp_h.at[pl.ds(sbase + off, SEG)])
        vs = stg_s[pl.ds(SEG, 16)]
        vd = stg_d[pl.ds(SEG, 16)]
        vw = stg_w[pl.ds(SEG, 16)]
        stg_s[pl.ds(0, 16)] = vs
        stg_d[pl.ds(0, 16)] = vd
        stg_w[pl.ds(0, 16)] = vw
        return staged - SEG, nseg + 1

    def p1_chunk(ci, carry):
        cnt, staged, nseg = carry
        base = pl.multiple_of(ebase + ci * C1, 8)
        pltpu.sync_copy(src_h.at[pl.ds(base, C1)], srcv)
        pltpu.sync_copy(dst_h.at[pl.ds(base, C1)], dstv)
        pltpu.sync_copy(w_h.at[pl.ds(base, C1)], wv)

        def step(j, carry):
            cnt, staged, nseg = carry
            off = pl.multiple_of(j * 16, 8)
            s16 = srcv[pl.ds(off, 16)]
            d16 = dstv[pl.ds(off, 16)]
            w16 = wv[pl.ds(off, 16)]
            msw = plsc.load_gather(midpv, [lax.shift_right_logical(s16, 2)])
            mdw = plsc.load_gather(midpv, [lax.shift_right_logical(d16, 2)])
            ms = lax.shift_right_logical(msw, (s16 & 3) * 8) & 0xFF
            md = lax.shift_right_logical(mdw, (d16 & 3) * 8) & 0xFF
            m = ms == md
            mi = m.astype(i32)
            pos = staged + plsc.cumsum(mi) - 1
            plsc.store_scatter(stg_s, [pos], s16, mask=m)
            plsc.store_scatter(stg_d, [pos], d16, mask=m)
            plsc.store_scatter(stg_w, [pos], w16, mask=m)
            tot = jnp.sum(mi)
            staged = staged + tot
            cnt = cnt + tot
            staged, nseg = lax.cond(staged >= SEG, _flush,
                                    lambda s, n: (s, n), staged, nseg)
            return cnt, staged, nseg

        return lax.fori_loop(0, C1 // 16, step, (cnt, staged, nseg))

    cnt, staged, nseg = lax.fori_loop(
        0, NCHUNK, p1_chunk, (jnp.int32(0), jnp.int32(0), jnp.int32(0)))

    for t in range(6):
        pos = staged + t * 16 + lane
        pm = pos < SEG + 16
        plsc.store_scatter(stg_s, [pos], z16i, mask=pm)
        plsc.store_scatter(stg_d, [pos], z16i, mask=pm)
        plsc.store_scatter(stg_w, [pos], z16f, mask=pm)
    _, nseg = _flush(staged, nseg)
    for t in range(SEG // 16):
        o = t * 16
        stg_s[pl.ds(o, 16)] = z16i
        stg_d[pl.ds(o, 16)] = z16i
        stg_w[pl.ds(o, 16)] = z16f
    _flush(jnp.int32(0), nseg)

    plsc.subcore_barrier()

    nch2 = (cnt + (C2 - 1)) // C2

    def p2_chunk(ci, _):
        cbase = pl.multiple_of(ci * C2, 8)
        pltpu.sync_copy(ssp_h.at[pl.ds(sbase + cbase, C2)], sidx)
        pltpu.sync_copy(dsp_h.at[pl.ds(sbase + cbase, C2)], didx)
        pltpu.sync_copy(wsp_h.at[pl.ds(sbase + cbase, C2)], wv2.at[pl.ds(0, C2)])
        cp1 = pltpu.async_copy(qtab.at[sidx], qb, sem)
        cp2 = pltpu.async_copy(kvtab.at[didx], kvb, sem)
        for j in range(C2 // 16):
            v = sidx[pl.ds(j * 16, 16)]
            sidx2[pl.ds(j * 16, 16)] = v
            sidx8[pl.ds(j * 16, 16)] = lax.shift_right_logical(v, 3)
        cp1.wait()
        cp2.wait()

        def edge(e, _):
            dvec = jnp.zeros((16,), f32)
            for h in range(NH):
                a = (qb[e, pl.ds(h * 32, 16)] * kvb[e, pl.ds(h * 32, 16)]
                     + qb[e, pl.ds(h * 32 + 16, 16)] * kvb[e, pl.ds(h * 32 + 16, 16)])
                dh = jnp.sum(a)
                dvec = jnp.where(lane == h, dh, dvec)
            w_e = wv2[pl.ds(e, 16)][0]
            z = w_e * wav + bav
            bias = 1.0 / (1.0 + jnp.exp(-z))
            lm = jnp.minimum(dvec * INV_SQRT_AD + bias, 50.0)
            keep = jnp.logical_and(lane4, (cbase + e) < cnt)
            exv = jnp.where(keep, jnp.exp(lm), 0.0)
            sel = sidx2[pl.ds(e, 16)][0]
            for k in range(D // 16):
                exb[e, pl.ds(k * 16, 16)] = z16f
            exb[e, pl.ds((sel & 7) * 16, 16)] = exv
            for h in range(NH):
                ah = exv[h]
                msgb[e, pl.ds(h * 32, 16)] = kvb[e, pl.ds(D + h * 32, 16)] * ah
                msgb[e, pl.ds(h * 32 + 16, 16)] = kvb[e, pl.ds(D + h * 32 + 16, 16)] * ah
            return 0

        lax.fori_loop(0, C2, edge, 0)
        pltpu.sync_copy(msgb, msgtab.at[sidx], add=True)
        pltpu.sync_copy(exb, extab.at[sidx8], add=True)
        return 0

    lax.fori_loop(0, nch2, p2_chunk, 0)

    plsc.subcore_barrier()
    rb2 = pl.multiple_of(sid * ROWS_EX_PT, 8)
    pltpu.sync_copy(msgtab.at[pl.ds(rbase, ROWS_PT)], omsg_h.at[cid, pl.ds(rbase, ROWS_PT)])
    pltpu.sync_copy(extab.at[pl.ds(rb2, ROWS_EX_PT)], oex_h.at[cid, pl.ds(rb2, ROWS_EX_PT)])


_edge_call = functools.partial(
    pl.kernel,
    out_type=(
        jax.ShapeDtypeStruct((NC, N_PAD, D), f32),
        jax.ShapeDtypeStruct((NC, NEX, D), f32),
        jax.ShapeDtypeStruct((NW * SPILL_W,), i32),
        jax.ShapeDtypeStruct((NW * SPILL_W,), i32),
        jax.ShapeDtypeStruct((NW * SPILL_W,), f32),
    ),
    mesh=plsc.VectorSubcoreMesh(core_axis_name="c", subcore_axis_name="s",
                                num_cores=NC, num_subcores=NS),
    compiler_params=pltpu.CompilerParams(needs_layout_passes=False),
    scratch_types=[
        pltpu.VMEM((C1,), i32),
        pltpu.VMEM((C1,), i32),
        pltpu.VMEM((C1,), f32),
        pltpu.VMEM((MODP,), i32),
        pltpu.VMEM((SEG + 16,), i32),
        pltpu.VMEM((SEG + 16,), i32),
        pltpu.VMEM((SEG + 16,), f32),
        pltpu.VMEM((C2,), i32),
        pltpu.VMEM((C2,), i32),
        pltpu.VMEM((C2 + 16,), i32),
        pltpu.VMEM((C2,), i32),
        pltpu.VMEM((C2 + 16,), f32),
        pltpu.VMEM((C2, D), f32),
        pltpu.VMEM((C2, 2 * D), f32),
        pltpu.VMEM((C2, D), f32),
        pltpu.VMEM((C2, D), f32),
        pltpu.VMEM((2, 16), f32),
        pltpu.VMEM_SHARED((N_PAD, D), f32),
        pltpu.VMEM_SHARED((NEX, D), f32),
        pltpu.SemaphoreType.DMA,
    ],
)(_edge_body)


def _ln(x, g, b):
    m = jnp.mean(x, axis=-1, keepdims=True)
    xc = x - m
    v = jnp.mean(xc * xc, axis=-1, keepdims=True)
    return xc * lax.rsqrt(v + 1e-5) * g + b


def _post_body(h_ref, ma_ref, mb_ref, ea_ref, eb_ref, s_ref, wo_ref,
               w1_ref, b1_ref, w2_ref, b2_ref, p1_ref, p2_ref, out_ref):
    num = ma_ref[...] + mb_ref[...]
    ex = ea_ref[...] + eb_ref[...]
    den = jnp.dot(ex, s_ref[...], preferred_element_type=f32)
    safe = jnp.where(den > 0.0, den, 1.0)
    agg = num / safe
    out = jnp.dot(agg, wo_ref[...], preferred_element_type=f32)
    x = h_ref[...] + out
    h1 = _ln(x, p1_ref[0:1, :], p1_ref[1:2, :])
    t = jnp.dot(h1, w1_ref[...], preferred_element_type=f32) + b1_ref[...]
    fmid = 0.5 * t * (1.0 + lax.erf(t * np.float32(1.0 / math.sqrt(2.0))))
    f2 = jnp.dot(fmid, w2_ref[...], preferred_element_type=f32) + b2_ref[...]
    out_ref[...] = _ln(h1 + f2, p2_ref[0:1, :], p2_ref[1:2, :])


_BR = N // 10
_post_call = pl.pallas_call(
    _post_body,
    grid=(10,),
    in_specs=[
        pl.BlockSpec((_BR, D), lambda i: (i, 0)),
        pl.BlockSpec((_BR, D), lambda i: (i, 0)),
        pl.BlockSpec((_BR, D), lambda i: (i, 0)),
        pl.BlockSpec((_BR, 16), lambda i: (i, 0)),
        pl.BlockSpec((_BR, 16), lambda i: (i, 0)),
        pl.BlockSpec((16, D), lambda i: (0, 0)),
        pl.BlockSpec((D, D), lambda i: (0, 0)),
        pl.BlockSpec((D, 4 * D), lambda i: (0, 0)),
        pl.BlockSpec((1, 4 * D), lambda i: (0, 0)),
        pl.BlockSpec((4 * D, D), lambda i: (0, 0)),
        pl.BlockSpec((1, D), lambda i: (0, 0)),
        pl.BlockSpec((2, D), lambda i: (0, 0)),
        pl.BlockSpec((2, D), lambda i: (0, 0)),
    ],
    out_specs=pl.BlockSpec((_BR, D), lambda i: (i, 0)),
    out_shape=jax.ShapeDtypeStruct((N, D), f32),
)

_S_NP = np.zeros((16, D), np.float32)
for _h in range(NH):
    _S_NP[_h, _h * AD:(_h + 1) * AD] = 1.0


def kernel(H, edge_index, edge_attr, module_id, Wq, Wk, Wv, Wo, w_a, b_a,
           ln1_g, ln1_b, W1, b1, W2, b2, ln2_g, ln2_b):
    Wqkv = jnp.concatenate([Wq, Wk, Wv], axis=1)
    qtab, kvtab = _qkv_call(H, Wqkv)
    src = edge_index[0]
    dst = edge_index[1]
    w = edge_attr.reshape(E)
    wab = jnp.stack([jnp.pad(w_a, (0, 12)), jnp.pad(b_a, (0, 12))])
    m4 = module_id.reshape(N // 4, 4)
    midp = (m4[:, 0] | (m4[:, 1] << 8) | (m4[:, 2] << 16) | (m4[:, 3] << 24))
    midp = jnp.pad(midp, (0, MODP - N // 4))
    omsg, oex, _, _, _ = _edge_call(qtab, kvtab, src, dst, w, midp, wab)
    omsg = omsg[:, :N, :]
    oex = oex.reshape(NC, N_PAD, 16)[:, :N, :]
    S = jnp.asarray(_S_NP)
    return _post_call(H, omsg[0], omsg[1], oex[0], oex[1], S, Wo,
                      W1, b1.reshape(1, 4 * D), W2, b2.reshape(1, D),
                      jnp.stack([ln1_g, ln1_b]), jnp.stack([ln2_g, ln2_b]))


# speedup vs baseline: 126.8891x; 1.4410x over previous
import functools
import math

import jax
import jax.numpy as jnp
import numpy as np
from jax import lax
from jax.experimental import pallas as pl
from jax.experimental.pallas import tpu as pltpu
from jax.experimental.pallas import tpu_sc as plsc

N = 10000
E = 320000
D = 128
NH = 4
AD = 32
INV_SQRT_AD = 1.0 / math.sqrt(AD)

NC = 2
NS = 16
NW = NC * NS
EPW = E // NW
C1 = 400
NCHUNK = EPW // C1
C2 = 48
SEG = 80
SPILL_W = EPW + 2 * SEG
MODP = 2504
N_PAD = 10240
ROWS_PT = N_PAD // NS
NEX = N_PAD // 8
ROWS_EX_PT = NEX // NS

f32 = jnp.float32
i32 = jnp.int32


def _qkv_body(h_ref, w_ref, q_ref, kv_ref):
    qkv = jnp.dot(h_ref[...], w_ref[...], preferred_element_type=f32)
    q_ref[...] = qkv[:, :D]
    kv_ref[...] = qkv[:, D:]


_qkv_call = pl.pallas_call(
    _qkv_body,
    grid=(10,),
    in_specs=[
        pl.BlockSpec((N // 10, D), lambda i: (i, 0)),
        pl.BlockSpec((D, 3 * D), lambda i: (0, 0)),
    ],
    out_specs=[
        pl.BlockSpec((N // 10, D), lambda i: (i, 0)),
        pl.BlockSpec((N // 10, 2 * D), lambda i: (i, 0)),
    ],
    out_shape=[
        jax.ShapeDtypeStruct((N, D), f32),
        jax.ShapeDtypeStruct((N, 2 * D), f32),
    ],
)


def _edge_body(
    qtab, kvtab, src_h, dst_h, w_h, midp_h, wab_h,
    omsg_h, oex_h, ssp_h, dsp_h, wsp_h,
    srcv0, srcv1, dstv0, dstv1, wv0, wv1, midpv,
    stg_s, stg_d, stg_w, sidx, didx, sidx2, sidx8, wv2,
    qb, kvb, msgb, exb, wabv,
    msgtab, extab,
    sem, sem1a, sem1b,
):
    cid = lax.axis_index("c")
    sid = lax.axis_index("s")
    wid = cid * NS + sid
    ebase = wid * EPW
    lane = lax.iota(i32, 16)
    lane4 = lane < 4
    z16i = jnp.zeros((16,), i32)
    z16f = jnp.zeros((16,), f32)

    rbase = pl.multiple_of(sid * ROWS_PT, 8)

    def zrow(r, _):
        for k in range(D // 16):
            msgb[r, pl.ds(k * 16, 16)] = z16f
            exb[r, pl.ds(k * 16, 16)] = z16f
        return 0

    lax.fori_loop(0, C2, zrow, 0)

    def zcp(t, _):
        ro = pl.multiple_of(rbase + t * 40, 8)
        pltpu.sync_copy(msgb.at[pl.ds(0, 40)], msgtab.at[pl.ds(ro, 40)])
        return 0

    lax.fori_loop(0, ROWS_PT // 40, zcp, 0)
    rbase2 = pl.multiple_of(sid * ROWS_EX_PT, 8)

    def zcp2(t, _):
        ro = pl.multiple_of(rbase2 + t * 40, 8)
        pltpu.sync_copy(exb.at[pl.ds(0, 40)], extab.at[pl.ds(ro, 40)])
        return 0

    lax.fori_loop(0, ROWS_EX_PT // 40, zcp2, 0)
    pltpu.sync_copy(wab_h, wabv)
    pltpu.sync_copy(midp_h, midpv)
    wav = wabv[0, :]
    bav = wabv[1, :]

    sbase = pl.multiple_of(wid * SPILL_W, 8)

    def _flush(staged, nseg):
        off = pl.multiple_of(nseg * SEG, 8)
        pltpu.sync_copy(stg_s.at[pl.ds(0, SEG)], ssp_h.at[pl.ds(sbase + off, SEG)])
        pltpu.sync_copy(stg_d.at[pl.ds(0, SEG)], dsp_h.at[pl.ds(sbase + off, SEG)])
        pltpu.sync_copy(stg_w.at[pl.ds(0, SEG)], wsp_h.at[pl.ds(sbase + off, SEG)])
        vs = stg_s[pl.ds(SEG, 16)]
        vd = stg_d[pl.ds(SEG, 16)]
        vw = stg_w[pl.ds(SEG, 16)]
        stg_s[pl.ds(0, 16)] = vs
        stg_d[pl.ds(0, 16)] = vd
        stg_w[pl.ds(0, 16)] = vw
        return staged - SEG, nseg + 1

    def _bufs(b):
        return (srcv0, dstv0, wv0, sem1a) if b == 0 else (srcv1, dstv1, wv1, sem1b)

    def _issue_p1(ci, b):
        base = pl.multiple_of(ebase + ci * C1, 8)
        sv, dv, wvb, sm = _bufs(b)
        return [
            pltpu.async_copy(src_h.at[pl.ds(base, C1)], sv, sm),
            pltpu.async_copy(dst_h.at[pl.ds(base, C1)], dv, sm),
            pltpu.async_copy(w_h.at[pl.ds(base, C1)], wvb, sm),
        ]

    def _make_step(b):
        sv, dv, wvb, _ = _bufs(b)

        def step(j, carry):
            cnt, staged, nseg = carry
            off = pl.multiple_of(j * 16, 8)
            s16 = sv[pl.ds(off, 16)]
            d16 = dv[pl.ds(off, 16)]
            w16 = wvb[pl.ds(off, 16)]
            msw = plsc.load_gather(midpv, [lax.shift_right_logical(s16, 2)])
            mdw = plsc.load_gather(midpv, [lax.shift_right_logical(d16, 2)])
            ms = lax.shift_right_logical(msw, (s16 & 3) * 8) & 0xFF
            md = lax.shift_right_logical(mdw, (d16 & 3) * 8) & 0xFF
            m = ms == md
            mi = m.astype(i32)
            pos = staged + plsc.cumsum(mi) - 1
            plsc.store_scatter(stg_s, [pos], s16, mask=m)
            plsc.store_scatter(stg_d, [pos], d16, mask=m)
            plsc.store_scatter(stg_w, [pos], w16, mask=m)
            tot = jnp.sum(mi)
            staged = staged + tot
            cnt = cnt + tot
            staged, nseg = lax.cond(staged >= SEG, _flush,
                                    lambda s, n: (s, n), staged, nseg)
            return cnt, staged, nseg
        return step

    carry = (jnp.int32(0), jnp.int32(0), jnp.int32(0))
    pend = _issue_p1(0, 0)
    for ci in range(NCHUNK):
        for cp in pend:
            cp.wait()
        if ci + 1 < NCHUNK:
            pend = _issue_p1(ci + 1, (ci + 1) % 2)
        carry = lax.fori_loop(0, C1 // 16, _make_step(ci % 2), carry)
    cnt, staged, nseg = carry

    for t in range(6):
        pos = staged + t * 16 + lane
        pm = pos < SEG + 16
        plsc.store_scatter(stg_s, [pos], z16i, mask=pm)
        plsc.store_scatter(stg_d, [pos], z16i, mask=pm)
        plsc.store_scatter(stg_w, [pos], z16f, mask=pm)
    _, nseg = _flush(staged, nseg)
    for t in range(SEG // 16):
        o = t * 16
        stg_s[pl.ds(o, 16)] = z16i
        stg_d[pl.ds(o, 16)] = z16i
        stg_w[pl.ds(o, 16)] = z16f
    _flush(jnp.int32(0), nseg)

    plsc.subcore_barrier()

    nch2 = (cnt + (C2 - 1)) // C2

    def p2_chunk(ci, _):
        cbase = pl.multiple_of(ci * C2, 8)
        ci1 = pltpu.async_copy(ssp_h.at[pl.ds(sbase + cbase, C2)], sidx, sem)
        ci2 = pltpu.async_copy(dsp_h.at[pl.ds(sbase + cbase, C2)], didx, sem)
        ci3 = pltpu.async_copy(wsp_h.at[pl.ds(sbase + cbase, C2)], wv2.at[pl.ds(0, C2)], sem)
        ci1.wait()
        ci2.wait()
        ci3.wait()
        cp1 = pltpu.async_copy(qtab.at[sidx], qb, sem)
        cp2 = pltpu.async_copy(kvtab.at[didx], kvb, sem)
        for j in range(C2 // 16):
            v = sidx[pl.ds(j * 16, 16)]
            sidx2[pl.ds(j * 16, 16)] = v
            sidx8[pl.ds(j * 16, 16)] = lax.shift_right_logical(v, 3)
        cp1.wait()
        cp2.wait()

        def edge(e, _):
            dvec = jnp.zeros((16,), f32)
            for h in range(NH):
                a = (qb[e, pl.ds(h * 32, 16)] * kvb[e, pl.ds(h * 32, 16)]
                     + qb[e, pl.ds(h * 32 + 16, 16)] * kvb[e, pl.ds(h * 32 + 16, 16)])
                dh = jnp.sum(a)
                dvec = jnp.where(lane == h, dh, dvec)
            w_e = wv2[pl.ds(e, 16)][0]
            z = w_e * wav + bav
            bias = 1.0 / (1.0 + jnp.exp(-z))
            lm = jnp.minimum(dvec * INV_SQRT_AD + bias, 50.0)
            keep = jnp.logical_and(lane4, (cbase + e) < cnt)
            exv = jnp.where(keep, jnp.exp(lm), 0.0)
            sel = sidx2[pl.ds(e, 16)][0]
            exb[e, pl.ds((sel & 7) * 16, 16)] = exv
            for h in range(NH):
                ah = exv[h]
                msgb[e, pl.ds(h * 32, 16)] = kvb[e, pl.ds(D + h * 32, 16)] * ah
                msgb[e, pl.ds(h * 32 + 16, 16)] = kvb[e, pl.ds(D + h * 32 + 16, 16)] * ah
            return 0

        lax.fori_loop(0, C2, edge, 0)
        pltpu.sync_copy(msgb, msgtab.at[sidx], add=True)
        pltpu.sync_copy(exb, extab.at[sidx8], add=True)

        def rezero(e, _):
            sel = sidx2[pl.ds(e, 16)][0]
            exb[e, pl.ds((sel & 7) * 16, 16)] = z16f
            return 0

        lax.fori_loop(0, C2, rezero, 0)
        return 0

    lax.fori_loop(0, nch2, p2_chunk, 0)

    plsc.subcore_barrier()
    rb2 = pl.multiple_of(sid * ROWS_EX_PT, 8)
    pltpu.sync_copy(msgtab.at[pl.ds(rbase, ROWS_PT)], omsg_h.at[cid, pl.ds(rbase, ROWS_PT)])
    pltpu.sync_copy(extab.at[pl.ds(rb2, ROWS_EX_PT)], oex_h.at[cid, pl.ds(rb2, ROWS_EX_PT)])


_edge_call = functools.partial(
    pl.kernel,
    out_type=(
        jax.ShapeDtypeStruct((NC, N_PAD, D), f32),
        jax.ShapeDtypeStruct((NC, NEX, D), f32),
        jax.ShapeDtypeStruct((NW * SPILL_W,), i32),
        jax.ShapeDtypeStruct((NW * SPILL_W,), i32),
        jax.ShapeDtypeStruct((NW * SPILL_W,), f32),
    ),
    mesh=plsc.VectorSubcoreMesh(core_axis_name="c", subcore_axis_name="s",
                                num_cores=NC, num_subcores=NS),
    compiler_params=pltpu.CompilerParams(needs_layout_passes=False),
    scratch_types=[
        pltpu.VMEM((C1,), i32),
        pltpu.VMEM((C1,), i32),
        pltpu.VMEM((C1,), i32),
        pltpu.VMEM((C1,), i32),
        pltpu.VMEM((C1,), f32),
        pltpu.VMEM((C1,), f32),
        pltpu.VMEM((MODP,), i32),
        pltpu.VMEM((SEG + 16,), i32),
        pltpu.VMEM((SEG + 16,), i32),
        pltpu.VMEM((SEG + 16,), f32),
        pltpu.VMEM((C2,), i32),
        pltpu.VMEM((C2,), i32),
        pltpu.VMEM((C2 + 16,), i32),
        pltpu.VMEM((C2,), i32),
        pltpu.VMEM((C2 + 16,), f32),
        pltpu.VMEM((C2, D), f32),
        pltpu.VMEM((C2, 2 * D), f32),
        pltpu.VMEM((C2, D), f32),
        pltpu.VMEM((C2, D), f32),
        pltpu.VMEM((2, 16), f32),
        pltpu.VMEM_SHARED((N_PAD, D), f32),
        pltpu.VMEM_SHARED((NEX, D), f32),
        pltpu.SemaphoreType.DMA,
        pltpu.SemaphoreType.DMA,
        pltpu.SemaphoreType.DMA,
    ],
)(_edge_body)


def _ln(x, g, b):
    m = jnp.mean(x, axis=-1, keepdims=True)
    xc = x - m
    v = jnp.mean(xc * xc, axis=-1, keepdims=True)
    return xc * lax.rsqrt(v + 1e-5) * g + b


def _post_body(h_ref, ma_ref, mb_ref, ea_ref, eb_ref, s_ref, wo_ref,
               w1_ref, b1_ref, w2_ref, b2_ref, p1_ref, p2_ref, out_ref):
    num = ma_ref[...] + mb_ref[...]
    ex = ea_ref[...] + eb_ref[...]
    den = jnp.dot(ex, s_ref[...], preferred_element_type=f32)
    safe = jnp.where(den > 0.0, den, 1.0)
    agg = num / safe
    out = jnp.dot(agg, wo_ref[...], preferred_element_type=f32)
    x = h_ref[...] + out
    h1 = _ln(x, p1_ref[0:1, :], p1_ref[1:2, :])
    t = jnp.dot(h1, w1_ref[...], preferred_element_type=f32) + b1_ref[...]
    fmid = 0.5 * t * (1.0 + lax.erf(t * np.float32(1.0 / math.sqrt(2.0))))
    f2 = jnp.dot(fmid, w2_ref[...], preferred_element_type=f32) + b2_ref[...]
    out_ref[...] = _ln(h1 + f2, p2_ref[0:1, :], p2_ref[1:2, :])


_BR = N // 10
_post_call = pl.pallas_call(
    _post_body,
    grid=(10,),
    in_specs=[
        pl.BlockSpec((_BR, D), lambda i: (i, 0)),
        pl.BlockSpec((_BR, D), lambda i: (i, 0)),
        pl.BlockSpec((_BR, D), lambda i: (i, 0)),
        pl.BlockSpec((_BR, 16), lambda i: (i, 0)),
        pl.BlockSpec((_BR, 16), lambda i: (i, 0)),
        pl.BlockSpec((16, D), lambda i: (0, 0)),
        pl.BlockSpec((D, D), lambda i: (0, 0)),
        pl.BlockSpec((D, 4 * D), lambda i: (0, 0)),
        pl.BlockSpec((1, 4 * D), lambda i: (0, 0)),
        pl.BlockSpec((4 * D, D), lambda i: (0, 0)),
        pl.BlockSpec((1, D), lambda i: (0, 0)),
        pl.BlockSpec((2, D), lambda i: (0, 0)),
        pl.BlockSpec((2, D), lambda i: (0, 0)),
    ],
    out_specs=pl.BlockSpec((_BR, D), lambda i: (i, 0)),
    out_shape=jax.ShapeDtypeStruct((N, D), f32),
)

_S_NP = np.zeros((16, D), np.float32)
for _h in range(NH):
    _S_NP[_h, _h * AD:(_h + 1) * AD] = 1.0


def kernel(H, edge_index, edge_attr, module_id, Wq, Wk, Wv, Wo, w_a, b_a,
           ln1_g, ln1_b, W1, b1, W2, b2, ln2_g, ln2_b):
    Wqkv = jnp.concatenate([Wq, Wk, Wv], axis=1)
    qtab, kvtab = _qkv_call(H, Wqkv)
    src = edge_index[0]
    dst = edge_index[1]
    w = edge_attr.reshape(E)
    wab = jnp.stack([jnp.pad(w_a, (0, 12)), jnp.pad(b_a, (0, 12))])
    m4 = module_id.reshape(N // 4, 4)
    midp = (m4[:, 0] | (m4[:, 1] << 8) | (m4[:, 2] << 16) | (m4[:, 3] << 24))
    midp = jnp.pad(midp, (0, MODP - N // 4))
    omsg, oex, _, _, _ = _edge_call(qtab, kvtab, src, dst, w, midp, wab)
    omsg = omsg[:, :N, :]
    oex = oex.reshape(NC, N_PAD, 16)[:, :N, :]
    S = jnp.asarray(_S_NP)
    return _post_call(H, omsg[0], omsg[1], oex[0], oex[1], S, Wo,
                      W1, b1.reshape(1, 4 * D), W2, b2.reshape(1, D),
                      jnp.stack([ln1_g, ln1_b]), jnp.stack([ln2_g, ln2_b]))


# speedup vs baseline: 165.3877x; 1.3034x over previous
import functools
import math

import jax
import jax.numpy as jnp
import numpy as np
from jax import lax
from jax.experimental import pallas as pl
from jax.experimental.pallas import tpu as pltpu
from jax.experimental.pallas import tpu_sc as plsc

N = 10000
E = 320000
D = 128
NH = 4
AD = 32
INV_SQRT_AD = 1.0 / math.sqrt(AD)

NC = 2
NS = 16
NW = NC * NS
EPW = E // NW
C1 = 400
NCHUNK = EPW // C1
C2 = 48
SEG = 80
SPILL_W = EPW + 2 * SEG
MODP = 2504
N_PAD = 10240
ROWS_PT = N_PAD // NS
NEX = N_PAD // 8
ROWS_EX_PT = NEX // NS

f32 = jnp.float32
i32 = jnp.int32


def _qkv_body(h_ref, w_ref, q_ref, kv_ref):
    qkv = jnp.dot(h_ref[...], w_ref[...], preferred_element_type=f32)
    q_ref[...] = qkv[:, :D]
    kv_ref[...] = qkv[:, D:]


_qkv_call = pl.pallas_call(
    _qkv_body,
    grid=(10,),
    in_specs=[
        pl.BlockSpec((N // 10, D), lambda i: (i, 0)),
        pl.BlockSpec((D, 3 * D), lambda i: (0, 0)),
    ],
    out_specs=[
        pl.BlockSpec((N // 10, D), lambda i: (i, 0)),
        pl.BlockSpec((N // 10, 2 * D), lambda i: (i, 0)),
    ],
    out_shape=[
        jax.ShapeDtypeStruct((N, D), f32),
        jax.ShapeDtypeStruct((N, 2 * D), f32),
    ],
)


def _edge_body(
    qtab, kvtab, src_h, dst_h, w_h, midp_h, wab_h,
    omsg_h, oex_h, ssp_h, dsp_h, wsp_h,
    srcv0, srcv1, dstv0, dstv1, wv0, wv1, midpv,
    stg_s, stg_d, stg_w, sidx, didx, sidx2, sidx8, wv2,
    qb, kvb, msgb, exb, wabv,
    msgtab, extab,
    sem, sem1a, sem1b,
):
    cid = lax.axis_index("c")
    sid = lax.axis_index("s")
    wid = cid * NS + sid
    ebase = wid * EPW
    lane = lax.iota(i32, 16)
    lane4 = lane < 4
    z16i = jnp.zeros((16,), i32)
    z16f = jnp.zeros((16,), f32)

    rbase = pl.multiple_of(sid * ROWS_PT, 8)

    def zrow(r, _):
        for k in range(D // 16):
            msgb[r, pl.ds(k * 16, 16)] = z16f
            exb[r, pl.ds(k * 16, 16)] = z16f
        return 0

    lax.fori_loop(0, C2, zrow, 0)

    def zcp(t, _):
        ro = pl.multiple_of(rbase + t * 40, 8)
        pltpu.sync_copy(msgb.at[pl.ds(0, 40)], msgtab.at[pl.ds(ro, 40)])
        return 0

    lax.fori_loop(0, ROWS_PT // 40, zcp, 0)
    rbase2 = pl.multiple_of(sid * ROWS_EX_PT, 8)

    def zcp2(t, _):
        ro = pl.multiple_of(rbase2 + t * 40, 8)
        pltpu.sync_copy(exb.at[pl.ds(0, 40)], extab.at[pl.ds(ro, 40)])
        return 0

    lax.fori_loop(0, ROWS_EX_PT // 40, zcp2, 0)
    pltpu.sync_copy(wab_h, wabv)
    pltpu.sync_copy(midp_h, midpv)
    wav = wabv[0, :]
    bav = wabv[1, :]

    sbase = pl.multiple_of(wid * SPILL_W, 8)

    def _flush(staged, nseg):
        off = pl.multiple_of(nseg * SEG, 8)
        pltpu.sync_copy(stg_s.at[pl.ds(0, SEG)], ssp_h.at[pl.ds(sbase + off, SEG)])
        pltpu.sync_copy(stg_d.at[pl.ds(0, SEG)], dsp_h.at[pl.ds(sbase + off, SEG)])
        pltpu.sync_copy(stg_w.at[pl.ds(0, SEG)], wsp_h.at[pl.ds(sbase + off, SEG)])
        vs = stg_s[pl.ds(SEG, 16)]
        vd = stg_d[pl.ds(SEG, 16)]
        vw = stg_w[pl.ds(SEG, 16)]
        stg_s[pl.ds(0, 16)] = vs
        stg_d[pl.ds(0, 16)] = vd
        stg_w[pl.ds(0, 16)] = vw
        return staged - SEG, nseg + 1

    def _bufs(b):
        return (srcv0, dstv0, wv0, sem1a) if b == 0 else (srcv1, dstv1, wv1, sem1b)

    def _issue_p1(ci, b):
        base = pl.multiple_of(ebase + ci * C1, 8)
        sv, dv, wvb, sm = _bufs(b)
        return [
            pltpu.async_copy(src_h.at[pl.ds(base, C1)], sv, sm),
            pltpu.async_copy(dst_h.at[pl.ds(base, C1)], dv, sm),
            pltpu.async_copy(w_h.at[pl.ds(base, C1)], wvb, sm),
        ]

    def _make_step(b):
        sv, dv, wvb, _ = _bufs(b)

        def step(j, carry):
            cnt, staged, nseg = carry
            off = pl.multiple_of(j * 16, 8)
            s16 = sv[pl.ds(off, 16)]
            d16 = dv[pl.ds(off, 16)]
            w16 = wvb[pl.ds(off, 16)]
            msw = plsc.load_gather(midpv, [lax.shift_right_logical(s16, 2)])
            mdw = plsc.load_gather(midpv, [lax.shift_right_logical(d16, 2)])
            ms = lax.shift_right_logical(msw, (s16 & 3) * 8) & 0xFF
            md = lax.shift_right_logical(mdw, (d16 & 3) * 8) & 0xFF
            m = ms == md
            mi = m.astype(i32)
            pos = staged + plsc.cumsum(mi) - 1
            plsc.store_scatter(stg_s, [pos], s16, mask=m)
            plsc.store_scatter(stg_d, [pos], d16, mask=m)
            plsc.store_scatter(stg_w, [pos], w16, mask=m)
            tot = jnp.sum(mi)
            staged = staged + tot
            cnt = cnt + tot
            staged, nseg = lax.cond(staged >= SEG, _flush,
                                    lambda s, n: (s, n), staged, nseg)
            return cnt, staged, nseg
        return step

    carry = (jnp.int32(0), jnp.int32(0), jnp.int32(0))
    pend = _issue_p1(0, 0)
    for ci in range(NCHUNK):
        for cp in pend:
            cp.wait()
        if ci + 1 < NCHUNK:
            pend = _issue_p1(ci + 1, (ci + 1) % 2)
        carry = lax.fori_loop(0, C1 // 16, _make_step(ci % 2), carry)
    cnt, staged, nseg = carry

    for t in range(6):
        pos = staged + t * 16 + lane
        pm = pos < SEG + 16
        plsc.store_scatter(stg_s, [pos], z16i, mask=pm)
        plsc.store_scatter(stg_d, [pos], z16i, mask=pm)
        plsc.store_scatter(stg_w, [pos], z16f, mask=pm)
    _, nseg = _flush(staged, nseg)
    for t in range(SEG // 16):
        o = t * 16
        stg_s[pl.ds(o, 16)] = z16i
        stg_d[pl.ds(o, 16)] = z16i
        stg_w[pl.ds(o, 16)] = z16f
    _flush(jnp.int32(0), nseg)

    plsc.subcore_barrier()

    nch2 = (cnt + (C2 - 1)) // C2

    def p2_chunk(ci, _):
        cbase = pl.multiple_of(ci * C2, 8)
        ci1 = pltpu.async_copy(ssp_h.at[pl.ds(sbase + cbase, C2)], sidx, sem)
        ci2 = pltpu.async_copy(dsp_h.at[pl.ds(sbase + cbase, C2)], didx, sem)
        ci3 = pltpu.async_copy(wsp_h.at[pl.ds(sbase + cbase, C2)], wv2.at[pl.ds(0, C2)], sem)
        ci1.wait()
        ci2.wait()
        ci3.wait()
        cp1 = pltpu.async_copy(qtab.at[sidx], qb, sem)
        cp2 = pltpu.async_copy(kvtab.at[didx], kvb, sem)
        for j in range(C2 // 16):
            v = sidx[pl.ds(j * 16, 16)]
            sidx2[pl.ds(j * 16, 16)] = v
            sidx8[pl.ds(j * 16, 16)] = lax.shift_right_logical(v, 3)
        cp1.wait()
        cp2.wait()

        @plsc.parallel_loop(0, C2, 1, unroll=2)
        def edge(e):
            dvec = jnp.zeros((16,), f32)
            for h in range(NH):
                a = (qb[e, pl.ds(h * 32, 16)] * kvb[e, pl.ds(h * 32, 16)]
                     + qb[e, pl.ds(h * 32 + 16, 16)] * kvb[e, pl.ds(h * 32 + 16, 16)])
                dh = jnp.sum(a)
                dvec = jnp.where(lane == h, dh, dvec)
            w_e = wv2[pl.ds(e, 16)][0]
            z = w_e * wav + bav
            bias = 1.0 / (1.0 + jnp.exp(-z))
            lm = jnp.minimum(dvec * INV_SQRT_AD + bias, 50.0)
            keep = jnp.logical_and(lane4, (cbase + e) < cnt)
            exv = jnp.where(keep, jnp.exp(lm), 0.0)
            sel = sidx2[pl.ds(e, 16)][0]
            exb[e, pl.ds((sel & 7) * 16, 16)] = exv
            for h in range(NH):
                ah = exv[h]
                msgb[e, pl.ds(h * 32, 16)] = kvb[e, pl.ds(D + h * 32, 16)] * ah
                msgb[e, pl.ds(h * 32 + 16, 16)] = kvb[e, pl.ds(D + h * 32 + 16, 16)] * ah

        pltpu.sync_copy(msgb, msgtab.at[sidx], add=True)
        pltpu.sync_copy(exb, extab.at[sidx8], add=True)

        @plsc.parallel_loop(0, C2, 1, unroll=4)
        def rezero(e):
            sel = sidx2[pl.ds(e, 16)][0]
            exb[e, pl.ds((sel & 7) * 16, 16)] = z16f
        return 0

    lax.fori_loop(0, nch2, p2_chunk, 0)

    plsc.subcore_barrier()
    rb2 = pl.multiple_of(sid * ROWS_EX_PT, 8)
    pltpu.sync_copy(msgtab.at[pl.ds(rbase, ROWS_PT)], omsg_h.at[cid, pl.ds(rbase, ROWS_PT)])
    pltpu.sync_copy(extab.at[pl.ds(rb2, ROWS_EX_PT)], oex_h.at[cid, pl.ds(rb2, ROWS_EX_PT)])


_edge_call = functools.partial(
    pl.kernel,
    out_type=(
        jax.ShapeDtypeStruct((NC, N_PAD, D), f32),
        jax.ShapeDtypeStruct((NC, NEX, D), f32),
        jax.ShapeDtypeStruct((NW * SPILL_W,), i32),
        jax.ShapeDtypeStruct((NW * SPILL_W,), i32),
        jax.ShapeDtypeStruct((NW * SPILL_W,), f32),
    ),
    mesh=plsc.VectorSubcoreMesh(core_axis_name="c", subcore_axis_name="s",
                                num_cores=NC, num_subcores=NS),
    compiler_params=pltpu.CompilerParams(needs_layout_passes=False),
    scratch_types=[
        pltpu.VMEM((C1,), i32),
        pltpu.VMEM((C1,), i32),
        pltpu.VMEM((C1,), i32),
        pltpu.VMEM((C1,), i32),
        pltpu.VMEM((C1,), f32),
        pltpu.VMEM((C1,), f32),
        pltpu.VMEM((MODP,), i32),
        pltpu.VMEM((SEG + 16,), i32),
        pltpu.VMEM((SEG + 16,), i32),
        pltpu.VMEM((SEG + 16,), f32),
        pltpu.VMEM((C2,), i32),
        pltpu.VMEM((C2,), i32),
        pltpu.VMEM((C2 + 16,), i32),
        pltpu.VMEM((C2,), i32),
        pltpu.VMEM((C2 + 16,), f32),
        pltpu.VMEM((C2, D), f32),
        pltpu.VMEM((C2, 2 * D), f32),
        pltpu.VMEM((C2, D), f32),
        pltpu.VMEM((C2, D), f32),
        pltpu.VMEM((2, 16), f32),
        pltpu.VMEM_SHARED((N_PAD, D), f32),
        pltpu.VMEM_SHARED((NEX, D), f32),
        pltpu.SemaphoreType.DMA,
        pltpu.SemaphoreType.DMA,
        pltpu.SemaphoreType.DMA,
    ],
)(_edge_body)


def _ln(x, g, b):
    m = jnp.mean(x, axis=-1, keepdims=True)
    xc = x - m
    v = jnp.mean(xc * xc, axis=-1, keepdims=True)
    return xc * lax.rsqrt(v + 1e-5) * g + b


def _post_body(h_ref, ma_ref, mb_ref, ea_ref, eb_ref, s_ref, wo_ref,
               w1_ref, b1_ref, w2_ref, b2_ref, p1_ref, p2_ref, out_ref):
    num = ma_ref[...] + mb_ref[...]
    ex = ea_ref[...] + eb_ref[...]
    den = jnp.dot(ex, s_ref[...], preferred_element_type=f32)
    safe = jnp.where(den > 0.0, den, 1.0)
    agg = num / safe
    out = jnp.dot(agg, wo_ref[...], preferred_element_type=f32)
    x = h_ref[...] + out
    h1 = _ln(x, p1_ref[0:1, :], p1_ref[1:2, :])
    t = jnp.dot(h1, w1_ref[...], preferred_element_type=f32) + b1_ref[...]
    fmid = 0.5 * t * (1.0 + lax.erf(t * np.float32(1.0 / math.sqrt(2.0))))
    f2 = jnp.dot(fmid, w2_ref[...], preferred_element_type=f32) + b2_ref[...]
    out_ref[...] = _ln(h1 + f2, p2_ref[0:1, :], p2_ref[1:2, :])


_BR = N // 10
_post_call = pl.pallas_call(
    _post_body,
    grid=(10,),
    in_specs=[
        pl.BlockSpec((_BR, D), lambda i: (i, 0)),
        pl.BlockSpec((_BR, D), lambda i: (i, 0)),
        pl.BlockSpec((_BR, D), lambda i: (i, 0)),
        pl.BlockSpec((_BR, 16), lambda i: (i, 0)),
        pl.BlockSpec((_BR, 16), lambda i: (i, 0)),
        pl.BlockSpec((16, D), lambda i: (0, 0)),
        pl.BlockSpec((D, D), lambda i: (0, 0)),
        pl.BlockSpec((D, 4 * D), lambda i: (0, 0)),
        pl.BlockSpec((1, 4 * D), lambda i: (0, 0)),
        pl.BlockSpec((4 * D, D), lambda i: (0, 0)),
        pl.BlockSpec((1, D), lambda i: (0, 0)),
        pl.BlockSpec((2, D), lambda i: (0, 0)),
        pl.BlockSpec((2, D), lambda i: (0, 0)),
    ],
    out_specs=pl.BlockSpec((_BR, D), lambda i: (i, 0)),
    out_shape=jax.ShapeDtypeStruct((N, D), f32),
)

_S_NP = np.zeros((16, D), np.float32)
for _h in range(NH):
    _S_NP[_h, _h * AD:(_h + 1) * AD] = 1.0


def kernel(H, edge_index, edge_attr, module_id, Wq, Wk, Wv, Wo, w_a, b_a,
           ln1_g, ln1_b, W1, b1, W2, b2, ln2_g, ln2_b):
    Wqkv = jnp.concatenate([Wq, Wk, Wv], axis=1)
    qtab, kvtab = _qkv_call(H, Wqkv)
    src = edge_index[0]
    dst = edge_index[1]
    w = edge_attr.reshape(E)
    wab = jnp.stack([jnp.pad(w_a, (0, 12)), jnp.pad(b_a, (0, 12))])
    m4 = module_id.reshape(N // 4, 4)
    midp = (m4[:, 0] | (m4[:, 1] << 8) | (m4[:, 2] << 16) | (m4[:, 3] << 24))
    midp = jnp.pad(midp, (0, MODP - N // 4))
    omsg, oex, _, _, _ = _edge_call(qtab, kvtab, src, dst, w, midp, wab)
    omsg = omsg[:, :N, :]
    oex = oex.reshape(NC, N_PAD, 16)[:, :N, :]
    S = jnp.asarray(_S_NP)
    return _post_call(H, omsg[0], omsg[1], oex[0], oex[1], S, Wo,
                      W1, b1.reshape(1, 4 * D), W2, b2.reshape(1, D),
                      jnp.stack([ln1_g, ln1_b]), jnp.stack([ln2_g, ln2_b]))
